# baseline (speedup 1.0000x reference)
import jax
import jax.numpy as jnp
from jax.experimental import pallas as pl
from jax.experimental.pallas import tpu as pltpu

_LANE = 128
_BATCH_TILE = 1024


def _pad_axis(a, axis, multiple):
    pad = (-a.shape[axis]) % multiple
    if pad == 0:
        return a
    widths = [(0, 0)] * a.ndim
    widths[axis] = (0, pad)
    return jnp.pad(a, widths)


def _mlp_kernel(x_ref, w1_ref, b1_ref, w2_ref, b2_ref, o_ref):
    xb = x_ref[...].astype(jnp.bfloat16)
    h = jnp.dot(xb, w1_ref[...], preferred_element_type=jnp.float32)
    h = jnp.maximum(h + b1_ref[...], 0.0).astype(jnp.bfloat16)
    out = jnp.dot(h, w2_ref[...], preferred_element_type=jnp.float32)
    o_ref[...] = (out + b2_ref[...]).astype(o_ref.dtype)


def kernel(x, w1, b1, w2, b2):
    B, IN = x.shape
    OUT = w2.shape[1]

    x_p = _pad_axis(x, 1, _LANE)
    w1_p = _pad_axis(_pad_axis(w1, 0, _LANE), 1, _LANE).astype(jnp.bfloat16)
    b1_p = _pad_axis(b1, 1, _LANE)
    w2_p = _pad_axis(_pad_axis(w2, 0, _LANE), 1, _LANE).astype(jnp.bfloat16)
    b2_p = _pad_axis(b2, 1, _LANE)
    IN_P, H_P = w1_p.shape
    OUT_P = w2_p.shape[1]

    tb = min(_BATCH_TILE, max(8, B))
    n_tiles = pl.cdiv(B, tb)
    x_p = _pad_axis(x_p, 0, tb)

    out_p = pl.pallas_call(
        _mlp_kernel,
        out_shape=jax.ShapeDtypeStruct((n_tiles * tb, OUT_P), x.dtype),
        grid=(n_tiles,),
        in_specs=[
            pl.BlockSpec((tb, IN_P), lambda i: (i, 0)),
            pl.BlockSpec((IN_P, H_P), lambda i: (0, 0)),
            pl.BlockSpec((1, H_P), lambda i: (0, 0)),
            pl.BlockSpec((H_P, OUT_P), lambda i: (0, 0)),
            pl.BlockSpec((1, OUT_P), lambda i: (0, 0)),
        ],
        out_specs=pl.BlockSpec((tb, OUT_P), lambda i: (i, 0)),
        compiler_params=pltpu.CompilerParams(
            dimension_semantics=("parallel",),
        ),
    )(x_p, w1_p, b1_p, w2_p, b2_p)
    return out_p[:B, :OUT]
